# Initial kernel scaffold; baseline (speedup 1.0000x reference)
import jax
import jax.numpy as jnp
from jax.experimental import pallas as pl


def kernel(x, W):
    raise NotImplementedError("write your pallas kernel here")


# speedup vs baseline: 1.8522x; 1.8522x over previous
import functools

import jax
import jax.numpy as jnp
from jax import lax
from jax.experimental import pallas as pl
from jax.experimental.pallas import tpu as pltpu

_N_TOKENS = 16384
_D_MODEL = 2048
_N_EXPERTS = 16
_TOP_K = 2
_BLK = 2048


def _router_body(x_ref, wt_ref, rw_ref, idx_ref, loss_ref, cnt_acc):
    step = pl.program_id(0)
    logits = jnp.dot(x_ref[...], wt_ref[...], preferred_element_type=jnp.float32)
    b = logits.shape[0]
    col = lax.broadcasted_iota(jnp.int32, (b, _N_EXPERTS), 1)

    m1 = jnp.max(logits, axis=-1, keepdims=True)
    i1 = jnp.min(jnp.where(logits == m1, col, _N_EXPERTS), axis=-1, keepdims=True)
    masked = jnp.where(col == i1, -jnp.inf, logits)
    m2 = jnp.max(masked, axis=-1, keepdims=True)
    i2 = jnp.min(jnp.where(masked == m2, col, _N_EXPERTS), axis=-1, keepdims=True)

    e2 = jnp.exp(m2 - m1)
    w1 = 1.0 / (1.0 + e2)
    rw_ref[...] = jnp.concatenate([w1, 1.0 - w1], axis=1)
    idx_ref[...] = jnp.concatenate([i1, i2], axis=1)

    onehot = (col == i1).astype(jnp.float32) + (col == i2).astype(jnp.float32)
    cnt = jnp.sum(onehot, axis=0, keepdims=True)

    @pl.when(step == 0)
    def _init():
        cnt_acc[...] = cnt

    @pl.when(step != 0)
    def _accum():
        cnt_acc[...] += cnt

    @pl.when(step == pl.num_programs(0) - 1)
    def _loss():
        c = cnt_acc[...]
        mean = jnp.sum(c) / _N_EXPERTS
        var = jnp.sum((c - mean) ** 2) / (_N_EXPERTS - 1)
        loss_ref[...] = (jnp.sqrt(var) / (mean + 1e-6) * 0.01).reshape(1, 1)


@functools.partial(jax.jit, static_argnames=())
def kernel(x, W):
    n, d = x.shape
    grid = n // _BLK
    rw, idx, loss = pl.pallas_call(
        _router_body,
        grid=(grid,),
        in_specs=[
            pl.BlockSpec((_BLK, d), lambda i: (i, 0)),
            pl.BlockSpec((d, _N_EXPERTS), lambda i: (0, 0)),
        ],
        out_specs=[
            pl.BlockSpec((_BLK, _TOP_K), lambda i: (i, 0)),
            pl.BlockSpec((_BLK, _TOP_K), lambda i: (i, 0)),
            pl.BlockSpec((1, 1), lambda i: (0, 0)),
        ],
        out_shape=[
            jax.ShapeDtypeStruct((n, _TOP_K), jnp.float32),
            jax.ShapeDtypeStruct((n, _TOP_K), jnp.int32),
            jax.ShapeDtypeStruct((1, 1), jnp.float32),
        ],
        scratch_shapes=[pltpu.VMEM((1, _N_EXPERTS), jnp.float32)],
        compiler_params=pltpu.CompilerParams(
            dimension_semantics=("arbitrary",),
        ),
    )(x, W.T)
    return rw, idx, loss.reshape(())
